# baseline (speedup 1.0000x reference)
import functools

import jax
import jax.numpy as jnp
from jax import lax
from jax.experimental import pallas as pl
from jax.experimental.pallas import tpu as pltpu
from jax.experimental.pallas import tpu_sc as plsc

N_IN = 100000
N_OUT = 100000
W = 128
LANES = 16
NW = 32
G = 128
GI = 2 * G
NCH_FULL = N_OUT // G
TAIL = N_OUT - NCH_FULL * G
NCH_HI = NCH_FULL // NW + 1
CUT = NCH_FULL - (NCH_HI - 1) * NW
HALF = NCH_HI * G
NBUF = 2


def _body(table, idx0, idx1, nmask, out,
          idx_v, mask_v,
          buf0, buf1, ob0, ob1,
          insem0, insem1, outsem0, outsem1):
    t = lax.axis_index("s") * 2 + lax.axis_index("c")
    nch = jnp.where(t < CUT, NCH_HI, NCH_HI - 1)
    c0 = NCH_HI * t - jnp.maximum(t - CUT, 0)

    m_base = pl.multiple_of(c0 * G, G)

    def stage(n):
        pltpu.sync_copy(idx0.at[pl.ds(m_base, n)], idx_v.at[pl.ds(0, n)])
        pltpu.sync_copy(idx1.at[pl.ds(m_base, n)], idx_v.at[pl.ds(HALF, n)])
        pltpu.sync_copy(nmask.at[pl.ds(m_base, n)], mask_v.at[pl.ds(0, n)])

    @pl.when(t < CUT)
    def _():
        stage(NCH_HI * G)

    @pl.when(jnp.logical_and(t >= CUT, t < NW - 1))
    def _():
        stage((NCH_HI - 1) * G)

    @pl.when(t == NW - 1)
    def _():
        stage((NCH_HI - 1) * G + TAIL)

    buf = (buf0, buf1)
    ob = (ob0, ob1)
    insem = (insem0, insem1)
    outsem = (outsem0, outsem1)

    def gather_parts(ci, s):
        a_off = pl.multiple_of(ci * G, G)
        return (
            (table.at[idx_v.at[pl.ds(a_off, G)]], buf[s].at[pl.ds(0, G)]),
            (table.at[idx_v.at[pl.ds(HALF + a_off, G)]],
             buf[s].at[pl.ds(G, G)]),
        )

    def start_gather(ci, s):
        for src_, dst in gather_parts(ci, s):
            pltpu.async_copy(src_, dst, insem[s])

    def wait_gather(ci, s):
        for src_, dst in gather_parts(ci, s):
            pltpu.make_async_copy(src_, dst, insem[s]).wait()

    def out_slice(ci):
        return out.at[pl.ds(pl.multiple_of((c0 + ci) * G, G), G)]

    def start_out(ci, s):
        pltpu.async_copy(ob[s], out_slice(ci), outsem[s])

    def wait_out(ci, s):
        pltpu.make_async_copy(ob[s], out_slice(ci), outsem[s]).wait()

    def compute(ci, s, ngroups=G // LANES):
        a_ref, o_ref = buf[s], ob[s]

        def g_body(g, carry):
            base = g * LANES
            m16 = mask_v[pl.ds(pl.multiple_of(ci * G + base, LANES), LANES)]
            for l in range(LANES):
                r = base + l
                m = jnp.full((LANES,), m16[l], jnp.int32)
                for w in range(W // LANES):
                    a = a_ref[r, pl.ds(w * LANES, LANES)]
                    b = a_ref[G + r, pl.ds(w * LANES, LANES)]
                    o_ref[r, pl.ds(w * LANES, LANES)] = (
                        ~((a & b) ^ (m & (a ^ b))))
            return carry

        lax.fori_loop(0, ngroups, g_body, 0)

    start_gather(0, 0)

    def ring_body(p, carry):
        for b in range(NBUF):
            ci = NBUF * p + b

            @pl.when(ci < nch)
            def _(ci=ci, b=b):
                s1 = (b + 1) % NBUF

                @pl.when(ci + 1 < nch)
                def _():
                    @pl.when(ci >= 1)
                    def _():
                        wait_out(ci - 1, s1)

                    start_gather(ci + 1, s1)

                wait_gather(ci, b)
                compute(ci, b)
                start_out(ci, b)
        return carry

    lax.fori_loop(0, (NCH_HI + NBUF - 1) // NBUF, ring_body, 0)
    for b in range(NBUF):
        last = nch - 1 - ((nch - 1 - b) % NBUF)
        wait_out(last, b)

    @pl.when(t == NW - 1)
    def _():
        tb = (NCH_HI - 1) * G
        pltpu.async_copy(
            table.at[idx_v.at[pl.ds(tb, TAIL)]],
            buf0.at[pl.ds(0, TAIL)], insem0)
        pltpu.async_copy(
            table.at[idx_v.at[pl.ds(HALF + tb, TAIL)]],
            buf0.at[pl.ds(G, TAIL)], insem0)
        pltpu.make_async_copy(
            table.at[idx_v.at[pl.ds(tb, TAIL)]],
            buf0.at[pl.ds(0, TAIL)], insem0).wait()
        pltpu.make_async_copy(
            table.at[idx_v.at[pl.ds(HALF + tb, TAIL)]],
            buf0.at[pl.ds(G, TAIL)], insem0).wait()
        compute(NCH_HI - 1, 0, ngroups=TAIL // LANES)
        pltpu.async_copy(ob0.at[pl.ds(0, TAIL)],
                         out.at[pl.ds(NCH_FULL * G, TAIL)], outsem0)
        pltpu.make_async_copy(ob0.at[pl.ds(0, TAIL)],
                              out.at[pl.ds(NCH_FULL * G, TAIL)],
                              outsem0).wait()


@jax.jit
def _nand_layer(table, idx0, idx1, nmask):
    mesh = plsc.VectorSubcoreMesh(core_axis_name="c", subcore_axis_name="s")
    f = functools.partial(
        pl.kernel,
        out_type=jax.ShapeDtypeStruct((N_OUT, W), jnp.int32),
        mesh=mesh,
        scratch_types=[
            pltpu.VMEM((NCH_HI * GI,), jnp.int32),
            pltpu.VMEM((NCH_HI * G,), jnp.int32),
            pltpu.VMEM((GI, W), jnp.int32),
            pltpu.VMEM((GI, W), jnp.int32),
            pltpu.VMEM((G, W), jnp.int32),
            pltpu.VMEM((G, W), jnp.int32),
            pltpu.SemaphoreType.DMA,
            pltpu.SemaphoreType.DMA,
            pltpu.SemaphoreType.DMA,
            pltpu.SemaphoreType.DMA,
        ],
    )(_body)
    return f(table, idx0, idx1, nmask)


def kernel(input_bitarrays, output_node_input_indices, nor_mask):
    idx = output_node_input_indices.astype(jnp.int32)
    nmask = jnp.where(nor_mask, jnp.int32(-1), jnp.int32(0))
    return _nand_layer(input_bitarrays, idx[:, 0], idx[:, 1], nmask)


# speedup vs baseline: 1.0554x; 1.0554x over previous
import functools

import jax
import jax.numpy as jnp
from jax import lax
from jax.experimental import pallas as pl
from jax.experimental.pallas import tpu as pltpu
from jax.experimental.pallas import tpu_sc as plsc

N_IN = 100000
N_OUT = 100000
W = 128
LANES = 16
NW = 32
G = 64
GI = 2 * G
NCH_FULL = N_OUT // G
TAIL = N_OUT - NCH_FULL * G
NCH_HI = 49
CUT = NCH_FULL - 48 * NW
HALF = NCH_HI * G
NBUF = 4


def _body(table, idx0, idx1, nmask, out,
          idx_v, mask_v,
          buf0, buf1, buf2, buf3, ob0, ob1, ob2, ob3,
          insem0, insem1, insem2, insem3,
          outsem0, outsem1, outsem2, outsem3):
    t = lax.axis_index("s") * 2 + lax.axis_index("c")
    nch = jnp.where(t < CUT, NCH_HI, NCH_HI - 1)
    c0 = NCH_HI * t - jnp.maximum(t - CUT, 0)

    m_base = pl.multiple_of(c0 * G, G)

    def stage(n):
        pltpu.sync_copy(idx0.at[pl.ds(m_base, n)], idx_v.at[pl.ds(0, n)])
        pltpu.sync_copy(idx1.at[pl.ds(m_base, n)], idx_v.at[pl.ds(HALF, n)])
        pltpu.sync_copy(nmask.at[pl.ds(m_base, n)], mask_v.at[pl.ds(0, n)])

    @pl.when(t < CUT)
    def _():
        stage(NCH_HI * G)

    @pl.when(jnp.logical_and(t >= CUT, t < NW - 1))
    def _():
        stage(48 * G)

    @pl.when(t == NW - 1)
    def _():
        stage(48 * G + TAIL)

    buf = (buf0, buf1, buf2, buf3)
    ob = (ob0, ob1, ob2, ob3)
    insem = (insem0, insem1, insem2, insem3)
    outsem = (outsem0, outsem1, outsem2, outsem3)

    def gather_parts(ci, s):
        a_off = pl.multiple_of(ci * G, G)
        return (
            (table.at[idx_v.at[pl.ds(a_off, G)]], buf[s].at[pl.ds(0, G)]),
            (table.at[idx_v.at[pl.ds(HALF + a_off, G)]],
             buf[s].at[pl.ds(G, G)]),
        )

    def start_gather(ci, s):
        for src_, dst in gather_parts(ci, s):
            pltpu.async_copy(src_, dst, insem[s])

    def wait_gather(ci, s):
        for src_, dst in gather_parts(ci, s):
            pltpu.make_async_copy(src_, dst, insem[s]).wait()

    def out_slice(ci):
        return out.at[pl.ds(pl.multiple_of((c0 + ci) * G, G), G)]

    def start_out(ci, s):
        pltpu.async_copy(ob[s], out_slice(ci), outsem[s])

    def wait_out(ci, s):
        pltpu.make_async_copy(ob[s], out_slice(ci), outsem[s]).wait()

    def compute(ci, s, ngroups=G // LANES):
        a_ref, o_ref = buf[s], ob[s]

        def g_body(g, carry):
            base = g * LANES
            m16 = mask_v[pl.ds(pl.multiple_of(ci * G + base, LANES), LANES)]
            for l in range(LANES):
                r = base + l
                m = jnp.full((LANES,), m16[l], jnp.int32)
                for w in range(W // LANES):
                    a = a_ref[r, pl.ds(w * LANES, LANES)]
                    b = a_ref[G + r, pl.ds(w * LANES, LANES)]
                    o_ref[r, pl.ds(w * LANES, LANES)] = (
                        ~((a & b) ^ (m & (a ^ b))))
            return carry

        lax.fori_loop(0, ngroups, g_body, 0)

    start_gather(0, 0)
    start_gather(1, 1)
    start_gather(2, 2)

    def ring_body(p, carry):
        for b in range(NBUF):
            ci = NBUF * p + b

            @pl.when(ci < nch)
            def _(ci=ci, b=b):
                s3 = (b + 3) % NBUF

                @pl.when(ci + 3 < nch)
                def _():
                    @pl.when(ci >= 1)
                    def _():
                        wait_out(ci - 1, s3)

                    start_gather(ci + 3, s3)

                wait_gather(ci, b)
                compute(ci, b)
                start_out(ci, b)
        return carry

    lax.fori_loop(0, (NCH_HI + NBUF - 1) // NBUF, ring_body, 0)
    for b in range(NBUF):
        last = nch - 1 - ((nch - 1 - b) % NBUF)
        wait_out(last, b)

    @pl.when(t == NW - 1)
    def _():
        pltpu.async_copy(
            table.at[idx_v.at[pl.ds(48 * G, TAIL)]],
            buf0.at[pl.ds(0, TAIL)], insem0)
        pltpu.async_copy(
            table.at[idx_v.at[pl.ds(HALF + 48 * G, TAIL)]],
            buf0.at[pl.ds(G, TAIL)], insem0)
        pltpu.make_async_copy(
            table.at[idx_v.at[pl.ds(48 * G, TAIL)]],
            buf0.at[pl.ds(0, TAIL)], insem0).wait()
        pltpu.make_async_copy(
            table.at[idx_v.at[pl.ds(HALF + 48 * G, TAIL)]],
            buf0.at[pl.ds(G, TAIL)], insem0).wait()
        compute(48, 0, ngroups=TAIL // LANES)
        pltpu.async_copy(ob0.at[pl.ds(0, TAIL)],
                         out.at[pl.ds(NCH_FULL * G, TAIL)], outsem0)
        pltpu.make_async_copy(ob0.at[pl.ds(0, TAIL)],
                              out.at[pl.ds(NCH_FULL * G, TAIL)],
                              outsem0).wait()


@jax.jit
def _nand_layer(table, idx0, idx1, nmask):
    mesh = plsc.VectorSubcoreMesh(core_axis_name="c", subcore_axis_name="s")
    f = functools.partial(
        pl.kernel,
        out_type=jax.ShapeDtypeStruct((N_OUT, W), jnp.int32),
        mesh=mesh,
        scratch_types=[
            pltpu.VMEM((NCH_HI * GI,), jnp.int32),
            pltpu.VMEM((NCH_HI * G,), jnp.int32),
            pltpu.VMEM((GI, W), jnp.int32),
            pltpu.VMEM((GI, W), jnp.int32),
            pltpu.VMEM((GI, W), jnp.int32),
            pltpu.VMEM((GI, W), jnp.int32),
            pltpu.VMEM((G, W), jnp.int32),
            pltpu.VMEM((G, W), jnp.int32),
            pltpu.VMEM((G, W), jnp.int32),
            pltpu.VMEM((G, W), jnp.int32),
            pltpu.SemaphoreType.DMA,
            pltpu.SemaphoreType.DMA,
            pltpu.SemaphoreType.DMA,
            pltpu.SemaphoreType.DMA,
            pltpu.SemaphoreType.DMA,
            pltpu.SemaphoreType.DMA,
            pltpu.SemaphoreType.DMA,
            pltpu.SemaphoreType.DMA,
        ],
    )(_body)
    return f(table, idx0, idx1, nmask)


def kernel(input_bitarrays, output_node_input_indices, nor_mask):
    idx = output_node_input_indices.astype(jnp.int32)
    nmask = jnp.where(nor_mask, jnp.int32(-1), jnp.int32(0))
    return _nand_layer(input_bitarrays, idx[:, 0], idx[:, 1], nmask)
